# baseline (speedup 1.0000x reference)
import functools

import jax
import jax.numpy as jnp
from jax import lax
from jax.experimental import pallas as pl
from jax.experimental.pallas import tpu as pltpu
from jax.experimental.pallas import tpu_sc as plsc

H = 1024
B = 4
LP = 256
LA = 2048
NL = 8
T_OUT = LP + 1 + LA + 1 + LA
NW = 32
CH = 64
NCH = LA // (4 * CH)

_mesh = plsc.VectorSubcoreMesh(core_axis_name="c", subcore_axis_name="s",
                               num_cores=2, num_subcores=16)


def _sc_body(acou_tab, pos_tab, phon_tab, sep_row, acou_idx, phon_idx,
             out, idx_v, pidx_v, rows_v, sem):
    wid = lax.axis_index("s") * 2 + lax.axis_index("c")

    block = wid // 4
    sub = wid % 4
    b = block // 2
    seg = block % 2
    chunk0 = block * 32 + sub * NCH
    dseg = 257 + seg * (LA + 1) + sub * (NCH * CH)
    for c in range(NCH):
        pltpu.sync_copy(acou_idx.at[chunk0 + c], idx_v)
        pltpu.async_copy(acou_tab.at[idx_v.at[0]], rows_v, sem).wait()
        for k in range(1, NL):
            pltpu.async_copy(acou_tab.at[idx_v.at[k]], rows_v, sem,
                             add=True).wait()
        pltpu.async_copy(pos_tab.at[idx_v.at[NL]], rows_v, sem,
                         add=True).wait()
        pltpu.sync_copy(rows_v, out.at[b, pl.ds(dseg + c * CH, CH)])

    prows = rows_v.at[pl.ds(0, 32)]
    pb = wid // 8
    pdst = (wid % 8) * 32
    pltpu.sync_copy(phon_idx.at[wid], pidx_v)
    pltpu.async_copy(phon_tab.at[pidx_v.at[0]], prows, sem).wait()
    pltpu.async_copy(pos_tab.at[pidx_v.at[1]], prows, sem, add=True).wait()
    pltpu.sync_copy(prows, out.at[pb, pl.ds(pdst, 32)])

    @pl.when(wid < 8)
    def _():
        srow = rows_v.at[pl.ds(0, 1)]
        sb = wid // 2
        sdst = 256 + (wid % 2) * (LA + 1)
        pltpu.sync_copy(sep_row, srow)
        pltpu.sync_copy(srow, out.at[sb, pl.ds(sdst, 1)])


_embed = pl.kernel(
    _sc_body,
    out_type=jax.ShapeDtypeStruct((B, T_OUT, H), jnp.float32),
    mesh=_mesh,
    scratch_types=[
        pltpu.VMEM((NL + 1, CH), jnp.int32),
        pltpu.VMEM((2, 32), jnp.int32),
        pltpu.VMEM((CH, H), jnp.float32),
        pltpu.SemaphoreType.DMA,
    ],
    compiler_params=pltpu.CompilerParams(use_tc_tiling_on_sc=False),
)


def _sinusoidal(max_len, dim):
    inv_freq = 1.0 / 10000 ** (jnp.arange(0, dim, 2, dtype=jnp.float32) / dim)
    t = jnp.arange(max_len, dtype=jnp.float32)
    ang = t[:, None] * inv_freq[None, :]
    return jnp.concatenate([jnp.sin(ang), jnp.cos(ang)], axis=-1)


@jax.jit
def kernel(phoneme_table, acoustic_tables, separator, phoneme_ids,
           prompt_ids, speech_ids):
    pos_tab = _sinusoidal(LA, H)
    acou_tab = acoustic_tables.reshape(NL * 1024, H)

    ids_all = jnp.stack([prompt_ids, speech_ids], axis=1)
    lev = ids_all.astype(jnp.int32) + (
        jnp.arange(NL, dtype=jnp.int32) * 1024)[None, None, :, None]
    lev = lev.transpose(0, 1, 3, 2).reshape(2 * B, LA, NL)
    pos_t = jnp.broadcast_to(
        jnp.arange(LA, dtype=jnp.int32)[None, :, None], (2 * B, LA, 1))
    allk = jnp.concatenate([lev, pos_t], axis=2)
    acou_idx = allk.reshape(2 * B, 32, CH, NL + 1)
    acou_idx = acou_idx.transpose(0, 1, 3, 2).reshape(2 * B * 32, NL + 1, CH)

    ph = phoneme_ids.astype(jnp.int32).reshape(NW, 32)
    pos_p = jnp.broadcast_to(
        jnp.arange(LP, dtype=jnp.int32).reshape(1, 8, 32),
        (B, 8, 32)).reshape(NW, 32)
    phon_idx = jnp.stack([ph, pos_p], axis=1)

    embeddings = _embed(acou_tab, pos_tab, phoneme_table,
                        separator.reshape(1, H), acou_idx, phon_idx)
    attention_mask = jnp.ones((B, T_OUT), dtype=jnp.float32)
    return (embeddings, attention_mask)


# speedup vs baseline: 1.3702x; 1.3702x over previous
import jax
import jax.numpy as jnp
from jax import lax
from jax.experimental import pallas as pl
from jax.experimental.pallas import tpu as pltpu
from jax.experimental.pallas import tpu_sc as plsc

H = 1024
B = 4
LP = 256
LA = 2048
NL = 8
T_OUT = LP + 1 + LA + 1 + LA
NW = 32
CH = 32
NCH = LA // (4 * CH)
NSRC = NL + 1
IDXW = NSRC * CH

_mesh = plsc.VectorSubcoreMesh(core_axis_name="c", subcore_axis_name="s",
                               num_cores=2, num_subcores=16)


def _accumulate(dst, src):
    @plsc.parallel_loop(0, CH)
    def _row(r):
        for u in range(H // 16):
            v = src[r, pl.ds(u * 16, 16)]
            plsc.addupdate(dst.at[r, pl.ds(u * 16, 16)], v)


def _sc_body(acou_tab, pos_tab, phon_tab, sep8, acou_idx, phon_idx, sep_dst,
             out, idx_v, pidx_v, sdst_v, rows_v, b0_v, b1_v,
             sem_r, sem_b, sem_s):
    wid = lax.axis_index("s") * 2 + lax.axis_index("c")
    lane = jnp.arange(16, dtype=jnp.int32)
    bufs = (b0_v, b1_v)

    block = wid // 4
    sub = wid % 4
    b = block // 2
    seg = block % 2
    chunk0 = block * (4 * NCH) + sub * NCH
    dst0 = b * T_OUT + 257 + seg * (LA + 1) + sub * (NCH * CH)

    def _chunk(c, _):
        pltpu.sync_copy(acou_idx.at[pl.ds((chunk0 + c) * IDXW, IDXW)], idx_v)
        g0 = pltpu.async_copy(acou_tab.at[idx_v.at[pl.ds(0, CH)]], rows_v,
                              sem_r)
        pend = pltpu.async_copy(acou_tab.at[idx_v.at[pl.ds(CH, CH)]], b0_v,
                                sem_b)
        g0.wait()
        for k in range(1, NSRC):
            if k + 1 < NSRC:
                tab = acou_tab if k + 1 < NSRC - 1 else pos_tab
                nxt = pltpu.async_copy(
                    tab.at[idx_v.at[pl.ds((k + 1) * CH, CH)]],
                    bufs[(k + 1) % 2], sem_b)
            else:
                nxt = None
            pend.wait()
            _accumulate(rows_v, bufs[k % 2])
            pend = nxt
        dchunk = dst0 + c * CH
        scat = [
            pltpu.async_copy(rows_v.at[pl.ds(q * 16, 16)],
                             out.at[lane + (dchunk + q * 16)], sem_s)
            for q in range(CH // 16)
        ]
        for d in scat:
            d.wait()
        return 0

    lax.fori_loop(0, NCH, _chunk, 0, unroll=False)

    pdst = (wid // 8) * T_OUT + (wid % 8) * 32
    pltpu.sync_copy(phon_idx.at[pl.ds(wid * 64, 64)], pidx_v)
    gp = pltpu.async_copy(phon_tab.at[pidx_v.at[pl.ds(0, 32)]], rows_v, sem_r)
    gq = pltpu.async_copy(pos_tab.at[pidx_v.at[pl.ds(32, 32)]], b0_v, sem_b)
    gp.wait()
    gq.wait()
    _accumulate(rows_v, b0_v)
    scat = [
        pltpu.async_copy(rows_v.at[pl.ds(q * 16, 16)],
                         out.at[lane + (pdst + q * 16)], sem_s)
        for q in range(2)
    ]
    for d in scat:
        d.wait()

    @pl.when(wid == 0)
    def _():
        srows = rows_v.at[pl.ds(0, 8)]
        pltpu.sync_copy(sep8, srows)
        pltpu.sync_copy(sep_dst, sdst_v)
        pltpu.async_copy(srows, out.at[sdst_v], sem_s).wait()


_embed = pl.kernel(
    _sc_body,
    out_type=jax.ShapeDtypeStruct((B * T_OUT, H), jnp.float32),
    mesh=_mesh,
    scratch_types=[
        pltpu.VMEM((IDXW,), jnp.int32),
        pltpu.VMEM((64,), jnp.int32),
        pltpu.VMEM((8,), jnp.int32),
        pltpu.VMEM((CH, H), jnp.float32),
        pltpu.VMEM((CH, H), jnp.float32),
        pltpu.VMEM((CH, H), jnp.float32),
        pltpu.SemaphoreType.DMA,
        pltpu.SemaphoreType.DMA,
        pltpu.SemaphoreType.DMA,
    ],
)


def _sinusoidal(max_len, dim):
    inv_freq = 1.0 / 10000 ** (jnp.arange(0, dim, 2, dtype=jnp.float32) / dim)
    t = jnp.arange(max_len, dtype=jnp.float32)
    ang = t[:, None] * inv_freq[None, :]
    return jnp.concatenate([jnp.sin(ang), jnp.cos(ang)], axis=-1)


@jax.jit
def kernel(phoneme_table, acoustic_tables, separator, phoneme_ids,
           prompt_ids, speech_ids):
    pos_tab = _sinusoidal(LA, H)
    acou_tab = acoustic_tables.reshape(NL * 1024, H)

    ids_all = jnp.stack([prompt_ids, speech_ids], axis=1)
    lev = ids_all.astype(jnp.int32) + (
        jnp.arange(NL, dtype=jnp.int32) * 1024)[None, None, :, None]
    lev = lev.transpose(0, 1, 3, 2).reshape(2 * B, LA, NL)
    pos_t = jnp.broadcast_to(
        jnp.arange(LA, dtype=jnp.int32)[None, :, None], (2 * B, LA, 1))
    allk = jnp.concatenate([lev, pos_t], axis=2)
    acou_idx = allk.reshape(2 * B, LA // CH, CH, NSRC)
    acou_idx = acou_idx.transpose(0, 1, 3, 2).reshape(2 * B * LA * NSRC)

    ph = phoneme_ids.astype(jnp.int32).reshape(NW, 32)
    pos_p = jnp.broadcast_to(
        jnp.arange(LP, dtype=jnp.int32).reshape(1, 8, 32),
        (B, 8, 32)).reshape(NW, 32)
    phon_idx = jnp.concatenate([ph, pos_p], axis=1).reshape(NW * 64)

    sep8 = jnp.broadcast_to(separator.reshape(1, H), (8, H))
    sep_dst = (jnp.arange(8, dtype=jnp.int32) // 2) * T_OUT + 256 + \
        (jnp.arange(8, dtype=jnp.int32) % 2) * (LA + 1)

    out_flat = _embed(acou_tab, pos_tab, phoneme_table, sep8,
                      acou_idx, phon_idx, sep_dst)
    embeddings = out_flat.reshape(B, T_OUT, H)
    attention_mask = jnp.ones((B, T_OUT), dtype=jnp.float32)
    return (embeddings, attention_mask)
